# Initial kernel scaffold; baseline (speedup 1.0000x reference)
import jax
import jax.numpy as jnp
from jax.experimental import pallas as pl


def kernel(node_feats, edge_feats, edge_index, batch_ids, solv_node_feats, solv_edge_index, solv_batch_ids, W_node, b_node, W_edge, b_edge, W_msg, b_msg, W_upd, b_upd, W_lin, b_lin, W_solv, b_solv, W_out, b_out):
    raise NotImplementedError("write your pallas kernel here")


# speedup vs baseline: 3.3678x; 3.3678x over previous
import functools

import jax
import jax.numpy as jnp
from jax import lax
from jax.experimental import pallas as pl
from jax.experimental.pallas import tpu as pltpu
from jax.experimental.pallas import tpu_sc as plsc

F32 = jnp.float32
N = 10000
E = 320000
D = 128
DE = 16
HID = 64
L = 4
G = 64
NS_NODES = 10000
ES = 160000

K = 128
SB = 4
NSC = 2
NSUB = 16
NW = NSC * NSUB
CPT = 80
CPT_S = 40
NCH = NW * CPT
NCH_S = NW * CPT_S
E_PAD = NCH * K
ES_PAD = NCH_S * K
NROWS = 10112
RPT = NROWS // NSUB


def _sc_edge_pass(hw, ep, src2d, dst2d, cpt, with_ep):
    nsteps = cpt // SB
    mesh = plsc.VectorSubcoreMesh(core_axis_name="c", subcore_axis_name="s")

    scratch = [
        pltpu.VMEM((cpt, K), jnp.int32),
        pltpu.VMEM((cpt, K), jnp.int32),
        pltpu.VMEM((SB * K, HID), F32),
    ]
    if with_ep:
        scratch.append(pltpu.VMEM((SB * K, HID), F32))
    scratch += [
        pltpu.VMEM_SHARED((NROWS, HID), F32),
        pltpu.SemaphoreType.DMA,
        pltpu.SemaphoreType.DMA,
    ]

    def body(*refs):
        if with_ep:
            (hw_hbm, ep_hbm, src_hbm, dst_hbm, out_hbm,
             src_all, dst_all, rows_v, m_v, agg_sh, sem_g, sem_e) = refs
        else:
            (hw_hbm, src_hbm, dst_hbm, out_hbm,
             src_all, dst_all, rows_v, agg_sh, sem_g, sem_e) = refs
            m_v = rows_v
        c = lax.axis_index("c")
        s = lax.axis_index("s")
        w = c * NSUB + s

        def zero_row(i, _):
            for jj in range(0, HID, 16):
                m_v[i, pl.ds(jj, 16)] = jnp.zeros((16,), F32)
            return 0
        lax.fori_loop(0, SB * K, zero_row, 0)
        base = s * RPT
        pltpu.sync_copy(m_v, agg_sh.at[pl.ds(base, SB * K)])
        rem = RPT - SB * K
        pltpu.sync_copy(m_v.at[pl.ds(0, rem)], agg_sh.at[pl.ds(base + SB * K, rem)])
        plsc.subcore_barrier()

        pltpu.sync_copy(src_hbm.at[pl.ds(w * cpt, cpt)], src_all)
        pltpu.sync_copy(dst_hbm.at[pl.ds(w * cpt, cpt)], dst_all)

        def step(t, _):
            q0 = w * cpt + t * SB
            descs = []
            for b in range(SB):
                descs.append(pltpu.async_copy(
                    hw_hbm.at[src_all.at[t * SB + b]],
                    rows_v.at[pl.ds(b * K, K)], sem_g))
            if with_ep:
                ep_desc = pltpu.async_copy(
                    ep_hbm.at[pl.ds(q0 * K, SB * K)], m_v, sem_e)
            for d in descs:
                d.wait()
            if with_ep:
                ep_desc.wait()

                def combine(i, _):
                    for jj in range(0, HID, 16):
                        m_v[i, pl.ds(jj, 16)] = jnp.maximum(
                            m_v[i, pl.ds(jj, 16)] + rows_v[i, pl.ds(jj, 16)],
                            jnp.zeros((16,), F32))
                    return 0
                lax.fori_loop(0, SB * K, combine, 0)
            for b in range(SB):
                pltpu.sync_copy(m_v.at[pl.ds(b * K, K)],
                                agg_sh.at[dst_all.at[t * SB + b]], add=True)
            return 0
        lax.fori_loop(0, nsteps, step, 0)

        plsc.subcore_barrier()
        pltpu.sync_copy(agg_sh.at[pl.ds(base, RPT)],
                        out_hbm.at[c, pl.ds(base, RPT)])

    run = functools.partial(
        pl.kernel,
        out_type=jax.ShapeDtypeStruct((NSC, NROWS, HID), F32),
        mesh=mesh,
        scratch_types=scratch,
        compiler_params=pltpu.CompilerParams(use_tc_tiling_on_sc=False),
    )(body)
    if with_ep:
        return run(hw, ep, src2d, dst2d)
    return run(hw, src2d, dst2d)


def _embed_body(nf_ref, snf_ref, wn_ref, bn_ref, a0_ref, ws_ref, bs_ref,
                h_ref, hw_ref, hs_ref):
    h0 = jnp.dot(nf_ref[...], wn_ref[...], preferred_element_type=F32) + bn_ref[...]
    h_ref[...] = h0
    hw_ref[...] = jnp.dot(h0, a0_ref[...], preferred_element_type=F32)
    hs_ref[...] = jnp.maximum(
        jnp.dot(snf_ref[...], ws_ref[...], preferred_element_type=F32) + bs_ref[...],
        0.0)


def _ep_body(ef_ref, we_ref, wb_ref, be_ref, bm_ref, out_ref):
    ef = ef_ref[...]
    for l in range(L):
        wb = wb_ref[l]
        m = jnp.dot(we_ref[...], wb, preferred_element_type=F32)
        cst = jnp.dot(be_ref[...], wb, preferred_element_type=F32) + bm_ref[l]
        out_ref[l] = jnp.dot(ef, m, preferred_element_type=F32) + cst


def _make_update(need_hw):
    def body(*refs):
        if need_hw:
            (h_ref, p_ref, ut_ref, ub_ref, b_ref, a_ref, ho_ref, hwo_ref) = refs
        else:
            (h_ref, p_ref, ut_ref, ub_ref, b_ref, ho_ref) = refs
        h = h_ref[...]
        agg = p_ref[0, :N, :] + p_ref[1, :N, :]
        u = jnp.maximum(
            jnp.dot(h, ut_ref[...], preferred_element_type=F32)
            + jnp.dot(agg, ub_ref[...], preferred_element_type=F32)
            + b_ref[...], 0.0)
        hn = h + u
        ho_ref[...] = hn
        if need_hw:
            hwo_ref[...] = jnp.dot(hn, a_ref[...], preferred_element_type=F32)
    return body


def _final_body(h_ref, hs0_ref, ps_ref, bi_ref, sbi_ref, wl_ref, bl_ref,
                wot_ref, wos_ref, bo_ref, out_ref):
    h = h_ref[...]
    hs = hs0_ref[...] + ps_ref[0, :NS_NODES, :] + ps_ref[1, :NS_NODES, :]
    gids = lax.broadcasted_iota(jnp.int32, (1, G), 1)
    oh = (bi_ref[...] == gids).astype(F32)
    ohs = (sbi_ref[...] == gids).astype(F32)
    g = lax.dot_general(oh, h, (((0,), (0,)), ((), ())),
                        preferred_element_type=F32)
    gs = lax.dot_general(ohs, hs, (((0,), (0,)), ((), ())),
                         preferred_element_type=F32)
    for l in range(2):
        g = jnp.maximum(
            jnp.dot(g, wl_ref[l], preferred_element_type=F32) + bl_ref[l], 0.0)
    out_ref[...] = (jnp.dot(g, wot_ref[...], preferred_element_type=F32)
                    + jnp.dot(gs, wos_ref[...], preferred_element_type=F32)
                    + bo_ref[...])


def kernel(node_feats, edge_feats, edge_index, batch_ids, solv_node_feats,
           solv_edge_index, solv_batch_ids, W_node, b_node, W_edge, b_edge,
           W_msg, b_msg, W_upd, b_upd, W_lin, b_lin, W_solv, b_solv,
           W_out, b_out):
    src = edge_index[0].astype(jnp.int32)
    dst = edge_index[1].astype(jnp.int32)
    pad = E_PAD - E
    pad_src = (jnp.arange(pad, dtype=jnp.int32) * 37) % N
    pad_dst = N + (jnp.arange(pad, dtype=jnp.int32) % (NROWS - N))
    src2d = jnp.concatenate([src, pad_src]).reshape(NCH, K)
    dst2d = jnp.concatenate([dst, pad_dst]).reshape(NCH, K)

    s_src = solv_edge_index[0].astype(jnp.int32)
    s_dst = solv_edge_index[1].astype(jnp.int32)
    pad_s = ES_PAD - ES
    pad_ssrc = (jnp.arange(pad_s, dtype=jnp.int32) * 37) % NS_NODES
    pad_sdst = N + (jnp.arange(pad_s, dtype=jnp.int32) % (NROWS - N))
    ssrc2d = jnp.concatenate([s_src, pad_ssrc]).reshape(NCH_S, K)
    sdst2d = jnp.concatenate([s_dst, pad_sdst]).reshape(NCH_S, K)

    ef_pad = jnp.concatenate(
        [edge_feats, jnp.zeros((pad, DE), F32)], axis=0)

    a_all = W_msg[:, :HID, :]
    wb_all = W_msg[:, HID:, :]
    ut_all = W_upd[:, :HID, :]
    ub_all = W_upd[:, HID:, :]

    h0, hw0, hs0 = pl.pallas_call(
        _embed_body,
        out_shape=[jax.ShapeDtypeStruct((N, HID), F32),
                   jax.ShapeDtypeStruct((N, HID), F32),
                   jax.ShapeDtypeStruct((NS_NODES, HID), F32)],
    )(node_feats, solv_node_feats, W_node, b_node.reshape(1, HID),
      a_all[0], W_solv, b_solv.reshape(1, HID))

    BE = 4096
    ep_all = pl.pallas_call(
        _ep_body,
        grid=(E_PAD // BE,),
        in_specs=[pl.BlockSpec((BE, DE), lambda i: (i, 0)),
                  pl.BlockSpec((DE, HID), lambda i: (0, 0)),
                  pl.BlockSpec((L, HID, HID), lambda i: (0, 0, 0)),
                  pl.BlockSpec((1, HID), lambda i: (0, 0)),
                  pl.BlockSpec((L, 1, HID), lambda i: (0, 0, 0))],
        out_specs=pl.BlockSpec((L, BE, HID), lambda i: (0, i, 0)),
        out_shape=jax.ShapeDtypeStruct((L, E_PAD, HID), F32),
    )(ef_pad, W_edge, wb_all, b_edge.reshape(1, HID),
      b_msg.reshape(L, 1, HID))

    h, hw = h0, hw0
    for l in range(L):
        part = _sc_edge_pass(hw, ep_all[l], src2d, dst2d, CPT, True)
        need_hw = l < L - 1
        outs = [jax.ShapeDtypeStruct((N, HID), F32)]
        args = [h, part, ut_all[l], ub_all[l], b_upd[l].reshape(1, HID)]
        if need_hw:
            outs.append(jax.ShapeDtypeStruct((N, HID), F32))
            args.append(a_all[l + 1])
        res = pl.pallas_call(_make_update(need_hw), out_shape=outs)(*args)
        if need_hw:
            h, hw = res
        else:
            h = res[0]

    ps = _sc_edge_pass(hs0, None, ssrc2d, sdst2d, CPT_S, False)

    out = pl.pallas_call(
        _final_body,
        out_shape=jax.ShapeDtypeStruct((G, 1), F32),
    )(h, hs0, ps, batch_ids.astype(jnp.int32).reshape(N, 1),
      solv_batch_ids.astype(jnp.int32).reshape(NS_NODES, 1),
      W_lin, b_lin.reshape(2, 1, HID), W_out[:HID], W_out[HID:],
      b_out.reshape(1, 1))
    return out


# speedup vs baseline: 4.9900x; 1.4817x over previous
import functools

import jax
import jax.numpy as jnp
from jax import lax
from jax.experimental import pallas as pl
from jax.experimental.pallas import tpu as pltpu
from jax.experimental.pallas import tpu_sc as plsc

F32 = jnp.float32
N = 10000
E = 320000
D = 128
DE = 16
HID = 64
L = 4
G = 64
NS_NODES = 10000
ES = 160000

K = 128
SB = 4
NSC = 2
NSUB = 16
NW = NSC * NSUB
CPT = 80
CPT_S = 40
NCH = NW * CPT
NCH_S = NW * CPT_S
E_PAD = NCH * K
ES_PAD = NCH_S * K
NROWS = 10112
RPT = NROWS // NSUB


def _sc_edge_pass(hw, ep, src2d, dst2d, cpt, with_ep):
    nsteps = cpt // SB
    mesh = plsc.VectorSubcoreMesh(core_axis_name="c", subcore_axis_name="s")

    scratch = [
        pltpu.VMEM((cpt, K), jnp.int32),
        pltpu.VMEM((cpt, K), jnp.int32),
        pltpu.VMEM((SB * K, HID), F32),
    ]
    if with_ep:
        scratch.append(pltpu.VMEM((SB * K // 2, 2 * HID), F32))
    scratch += [
        pltpu.VMEM_SHARED((NROWS, HID), F32),
        pltpu.SemaphoreType.DMA,
        pltpu.SemaphoreType.DMA,
    ]

    def body(*refs):
        if with_ep:
            (hw_hbm, ep_hbm, src_hbm, dst_hbm, out_hbm,
             src_all, dst_all, rows_v, ep_v, agg_sh, sem_g, sem_e) = refs
        else:
            (hw_hbm, src_hbm, dst_hbm, out_hbm,
             src_all, dst_all, rows_v, agg_sh, sem_g, sem_e) = refs
        m_v = rows_v
        c = lax.axis_index("c")
        s = lax.axis_index("s")
        w = c * NSUB + s

        def zero_row(i, _):
            for jj in range(0, HID, 16):
                m_v[i, pl.ds(jj, 16)] = jnp.zeros((16,), F32)
            return 0
        lax.fori_loop(0, SB * K, zero_row, 0)
        base = s * RPT
        pltpu.sync_copy(m_v, agg_sh.at[pl.ds(base, SB * K)])
        rem = RPT - SB * K
        pltpu.sync_copy(m_v.at[pl.ds(0, rem)], agg_sh.at[pl.ds(base + SB * K, rem)])
        plsc.subcore_barrier()

        pltpu.sync_copy(src_hbm.at[pl.ds(w * cpt, cpt)], src_all)
        pltpu.sync_copy(dst_hbm.at[pl.ds(w * cpt, cpt)], dst_all)

        def step(t, _):
            q0 = w * cpt + t * SB
            descs = []
            for b in range(SB):
                descs.append(pltpu.async_copy(
                    hw_hbm.at[src_all.at[t * SB + b]],
                    rows_v.at[pl.ds(b * K, K)], sem_g))
            if with_ep:
                ep_desc = pltpu.async_copy(
                    ep_hbm.at[pl.ds(q0 * (K // 2), SB * K // 2)], ep_v, sem_e)
            for d in descs:
                d.wait()
            if with_ep:
                ep_desc.wait()

                def combine(p, _):
                    for h in range(2):
                        for jj in range(0, HID, 16):
                            m_v[2 * p + h, pl.ds(jj, 16)] = jnp.maximum(
                                rows_v[2 * p + h, pl.ds(jj, 16)]
                                + ep_v[p, pl.ds(h * HID + jj, 16)],
                                jnp.zeros((16,), F32))
                    return 0
                lax.fori_loop(0, SB * K // 2, combine, 0)
            for b in range(SB):
                pltpu.sync_copy(m_v.at[pl.ds(b * K, K)],
                                agg_sh.at[dst_all.at[t * SB + b]], add=True)
            return 0
        lax.fori_loop(0, nsteps, step, 0)

        plsc.subcore_barrier()
        pltpu.sync_copy(agg_sh.at[pl.ds(base, RPT)],
                        out_hbm.at[c, pl.ds(base, RPT)])

    run = functools.partial(
        pl.kernel,
        out_type=jax.ShapeDtypeStruct((NSC, NROWS, HID), F32),
        mesh=mesh,
        scratch_types=scratch,
        compiler_params=pltpu.CompilerParams(use_tc_tiling_on_sc=False),
    )(body)
    if with_ep:
        return run(hw, ep, src2d, dst2d)
    return run(hw, src2d, dst2d)


def _embed_body(nf_ref, snf_ref, wn_ref, bn_ref, a0_ref, ws_ref, bs_ref,
                h_ref, hw_ref, hs_ref):
    h0 = jnp.dot(nf_ref[...], wn_ref[...], preferred_element_type=F32) + bn_ref[...]
    h_ref[...] = h0
    hw_ref[...] = jnp.dot(h0, a0_ref[...], preferred_element_type=F32)
    hs_ref[...] = jnp.maximum(
        jnp.dot(snf_ref[...], ws_ref[...], preferred_element_type=F32) + bs_ref[...],
        0.0)


def _ep_body(ef_ref, we_ref, wb_ref, be_ref, bm_ref, *out_refs):
    ef = ef_ref[...]
    z = jnp.zeros((DE, HID), F32)
    for l in range(L):
        wb = wb_ref[l]
        m = jnp.dot(we_ref[...], wb, preferred_element_type=F32)
        m2 = jnp.concatenate(
            [jnp.concatenate([m, z], axis=1),
             jnp.concatenate([z, m], axis=1)], axis=0)
        cst = jnp.dot(be_ref[...], wb, preferred_element_type=F32) + bm_ref[l]
        cst2 = jnp.concatenate([cst, cst], axis=1)
        out_refs[l][...] = jnp.dot(ef, m2, preferred_element_type=F32) + cst2


def _make_update(need_hw):
    def body(*refs):
        if need_hw:
            (h_ref, p_ref, ut_ref, ub_ref, b_ref, a_ref, ho_ref, hwo_ref) = refs
        else:
            (h_ref, p_ref, ut_ref, ub_ref, b_ref, ho_ref) = refs
        h = h_ref[...]
        agg = p_ref[0, :N, :] + p_ref[1, :N, :]
        u = jnp.maximum(
            jnp.dot(h, ut_ref[...], preferred_element_type=F32)
            + jnp.dot(agg, ub_ref[...], preferred_element_type=F32)
            + b_ref[...], 0.0)
        hn = h + u
        ho_ref[...] = hn
        if need_hw:
            hwo_ref[...] = jnp.dot(hn, a_ref[...], preferred_element_type=F32)
    return body


def _final_body(h_ref, hs0_ref, ps_ref, bi_ref, sbi_ref, wl_ref, bl_ref,
                wot_ref, wos_ref, bo_ref, out_ref):
    h = h_ref[...]
    hs = hs0_ref[...] + ps_ref[0, :NS_NODES, :] + ps_ref[1, :NS_NODES, :]
    gids = lax.broadcasted_iota(jnp.int32, (1, G), 1)
    oh = (bi_ref[...] == gids).astype(F32)
    ohs = (sbi_ref[...] == gids).astype(F32)
    g = lax.dot_general(oh, h, (((0,), (0,)), ((), ())),
                        preferred_element_type=F32)
    gs = lax.dot_general(ohs, hs, (((0,), (0,)), ((), ())),
                         preferred_element_type=F32)
    for l in range(2):
        g = jnp.maximum(
            jnp.dot(g, wl_ref[l], preferred_element_type=F32) + bl_ref[l], 0.0)
    out_ref[...] = (jnp.dot(g, wot_ref[...], preferred_element_type=F32)
                    + jnp.dot(gs, wos_ref[...], preferred_element_type=F32)
                    + bo_ref[...])


def kernel(node_feats, edge_feats, edge_index, batch_ids, solv_node_feats,
           solv_edge_index, solv_batch_ids, W_node, b_node, W_edge, b_edge,
           W_msg, b_msg, W_upd, b_upd, W_lin, b_lin, W_solv, b_solv,
           W_out, b_out):
    src = edge_index[0].astype(jnp.int32)
    dst = edge_index[1].astype(jnp.int32)
    pad = E_PAD - E
    pad_src = (jnp.arange(pad, dtype=jnp.int32) * 37) % N
    pad_dst = N + (jnp.arange(pad, dtype=jnp.int32) % (NROWS - N))
    src2d = jnp.concatenate([src, pad_src]).reshape(NCH, K)
    dst2d = jnp.concatenate([dst, pad_dst]).reshape(NCH, K)

    s_src = solv_edge_index[0].astype(jnp.int32)
    s_dst = solv_edge_index[1].astype(jnp.int32)
    pad_s = ES_PAD - ES
    pad_ssrc = (jnp.arange(pad_s, dtype=jnp.int32) * 37) % NS_NODES
    pad_sdst = N + (jnp.arange(pad_s, dtype=jnp.int32) % (NROWS - N))
    ssrc2d = jnp.concatenate([s_src, pad_ssrc]).reshape(NCH_S, K)
    sdst2d = jnp.concatenate([s_dst, pad_sdst]).reshape(NCH_S, K)

    ef_pair = jnp.concatenate(
        [edge_feats, jnp.zeros((pad, DE), F32)], axis=0).reshape(E_PAD // 2, 2 * DE)

    a_all = W_msg[:, :HID, :]
    wb_all = W_msg[:, HID:, :]
    ut_all = W_upd[:, :HID, :]
    ub_all = W_upd[:, HID:, :]

    h0, hw0, hs0 = pl.pallas_call(
        _embed_body,
        out_shape=[jax.ShapeDtypeStruct((N, HID), F32),
                   jax.ShapeDtypeStruct((N, HID), F32),
                   jax.ShapeDtypeStruct((NS_NODES, HID), F32)],
    )(node_feats, solv_node_feats, W_node, b_node.reshape(1, HID),
      a_all[0], W_solv, b_solv.reshape(1, HID))

    BE = 2048
    EPH = E_PAD // 2
    ep_all = pl.pallas_call(
        _ep_body,
        grid=(EPH // BE,),
        in_specs=[pl.BlockSpec((BE, 2 * DE), lambda i: (i, 0)),
                  pl.BlockSpec((DE, HID), lambda i: (0, 0)),
                  pl.BlockSpec((L, HID, HID), lambda i: (0, 0, 0)),
                  pl.BlockSpec((1, HID), lambda i: (0, 0)),
                  pl.BlockSpec((L, 1, HID), lambda i: (0, 0, 0))],
        out_specs=[pl.BlockSpec((BE, 2 * HID), lambda i: (i, 0))] * L,
        out_shape=[jax.ShapeDtypeStruct((EPH, 2 * HID), F32)] * L,
    )(ef_pair, W_edge, wb_all, b_edge.reshape(1, HID),
      b_msg.reshape(L, 1, HID))

    h, hw = h0, hw0
    for l in range(L):
        part = _sc_edge_pass(hw, ep_all[l], src2d, dst2d, CPT, True)
        need_hw = l < L - 1
        outs = [jax.ShapeDtypeStruct((N, HID), F32)]
        args = [h, part, ut_all[l], ub_all[l], b_upd[l].reshape(1, HID)]
        if need_hw:
            outs.append(jax.ShapeDtypeStruct((N, HID), F32))
            args.append(a_all[l + 1])
        res = pl.pallas_call(_make_update(need_hw), out_shape=outs)(*args)
        if need_hw:
            h, hw = res
        else:
            h = res[0]

    ps = _sc_edge_pass(hs0, None, ssrc2d, sdst2d, CPT_S, False)

    out = pl.pallas_call(
        _final_body,
        out_shape=jax.ShapeDtypeStruct((G, 1), F32),
    )(h, hs0, ps, batch_ids.astype(jnp.int32).reshape(N, 1),
      solv_batch_ids.astype(jnp.int32).reshape(NS_NODES, 1),
      W_lin, b_lin.reshape(2, 1, HID), W_out[:HID], W_out[HID:],
      b_out.reshape(1, 1))
    return out
